# baseline (speedup 1.0000x reference)
import functools

import jax
import jax.numpy as jnp
from jax import lax
from jax.experimental import pallas as pl
from jax.experimental.pallas import tpu as pltpu
from jax.experimental.pallas import tpu_sc as plsc

NUM_CORES = 2
NUM_SUBCORES = 16
NUM_WORKERS = NUM_CORES * NUM_SUBCORES
LANES = 16
CHUNK = 64


def kernel(h, r, t, entity_table, relation_table):
    batch = h.shape[0]
    ecount, dim = entity_table.shape
    assert batch % (8 * NUM_WORKERS) == 0 and dim == 64
    b_per_w = batch // NUM_WORKERS
    n_chunks = b_per_w // CHUNK

    etab2 = entity_table.reshape(ecount // 2, 2 * dim)
    rtab2 = relation_table.reshape(ecount // 2, 2 * dim)

    mesh = plsc.VectorSubcoreMesh(core_axis_name="c", subcore_axis_name="s")
    out_sds = jax.ShapeDtypeStruct((batch, dim), jnp.float32)

    @functools.partial(
        pl.kernel,
        out_type=(out_sds, out_sds, out_sds),
        mesh=mesh,
        compiler_params=pltpu.CompilerParams(needs_layout_passes=False),
        scratch_types=[
            pltpu.VMEM((b_per_w,), jnp.int32),
            pltpu.VMEM((b_per_w,), jnp.int32),
            pltpu.VMEM((CHUNK, 2 * dim), jnp.float32),
            pltpu.VMEM((CHUNK, dim), jnp.float32),
            pltpu.SemaphoreType.DMA,
        ],
    )
    def emb_kernel(h_hbm, r_hbm, t_hbm, etab, rtab, h_out, r_out, t_out,
                   idx_v, pid_v, packed_v, rows_v, sem):
        wid = lax.axis_index("s") * NUM_CORES + lax.axis_index("c")
        base = wid * b_per_w
        lane = lax.iota(jnp.int32, LANES)

        def run_lookup(src_hbm, table, out_hbm):
            pltpu.sync_copy(src_hbm.at[pl.ds(base, b_per_w)], idx_v)

            def pid_body(k, _):
                off = pl.ds(k * LANES, LANES)
                pid_v[off] = lax.shift_right_logical(idx_v[off], 1)
                return 0
            lax.fori_loop(0, b_per_w // LANES, pid_body, 0)

            def chunk_body(c, _):
                cb = c * CHUNK
                pltpu.async_copy(
                    table.at[pid_v.at[pl.ds(cb, CHUNK)]], packed_v, sem
                ).wait()
                for g in range(CHUNK // LANES):
                    jv = lane + g * LANES
                    ivec = idx_v[pl.ds(cb + g * LANES, LANES)]
                    bv = lax.shift_left(lax.bitwise_and(ivec, 1), 6)
                    def col_body(q, _):
                        cv = jnp.full((LANES,), q, jnp.int32)
                        vals = plsc.load_gather(packed_v, [jv, bv + cv])
                        plsc.store_scatter(rows_v, [jv, cv], vals)
                        return 0
                    lax.fori_loop(0, dim, col_body, 0)
                pltpu.sync_copy(rows_v, out_hbm.at[pl.ds(base + cb, CHUNK)])
                return 0
            lax.fori_loop(0, n_chunks, chunk_body, 0)

        run_lookup(h_hbm, etab, h_out)
        run_lookup(r_hbm, rtab, r_out)
        run_lookup(t_hbm, etab, t_out)

    return emb_kernel(h, r, t, etab2, rtab2)


# speedup vs baseline: 1.5914x; 1.5914x over previous
import functools

import jax
import jax.numpy as jnp
from jax import lax
from jax.experimental import pallas as pl
from jax.experimental.pallas import tpu as pltpu
from jax.experimental.pallas import tpu_sc as plsc

NUM_CORES = 2
NUM_SUBCORES = 16
NUM_WORKERS = NUM_CORES * NUM_SUBCORES
CHUNK = 32
HALF = CHUNK // 2


def kernel(h, r, t, entity_table, relation_table):
    batch = h.shape[0]
    dim = entity_table.shape[1]
    assert batch % (8 * NUM_WORKERS) == 0
    b_per_w = batch // NUM_WORKERS
    n_chunks = b_per_w // CHUNK

    mesh = plsc.VectorSubcoreMesh(core_axis_name="c", subcore_axis_name="s")
    out_sds = jax.ShapeDtypeStruct((batch, dim), jnp.float32)

    @functools.partial(
        pl.kernel,
        out_type=(out_sds, out_sds, out_sds),
        mesh=mesh,
        scratch_types=[
            pltpu.VMEM((b_per_w,), jnp.int32),
            pltpu.VMEM((HALF, dim), jnp.float32),
            pltpu.VMEM_SHARED((NUM_SUBCORES, HALF, dim), jnp.float32),
            pltpu.SemaphoreType.DMA,
            pltpu.SemaphoreType.DMA,
        ],
    )
    def emb_kernel(h_hbm, r_hbm, t_hbm, etab, rtab, h_out, r_out, t_out,
                   idx_v, rows_v, srows_v, sem_a, sem_b):
        cid = lax.axis_index("c")
        sid = lax.axis_index("s")
        wid = sid * NUM_CORES + cid
        base = wid * b_per_w
        my_srows = srows_v.at[sid]

        def run_lookup(src_hbm, table, out_hbm):
            pltpu.sync_copy(src_hbm.at[pl.ds(base, b_per_w)], idx_v)

            def chunk_body(c, _):
                cb = c * CHUNK
                copies = []
                for g in range(CHUNK // 16):
                    ivec = idx_v[pl.ds(cb + g * 16, 16)]
                    for k in range(16):
                        slot = g * 16 + k
                        if slot < HALF:
                            copies.append(pltpu.async_copy(
                                table.at[ivec[k]], rows_v.at[slot], sem_a))
                        else:
                            copies.append(pltpu.async_copy(
                                table.at[ivec[k]], my_srows.at[slot - HALF],
                                sem_b))
                for cp in copies:
                    cp.wait()
                pltpu.sync_copy(rows_v, out_hbm.at[pl.ds(base + cb, HALF)])
                pltpu.sync_copy(my_srows,
                                out_hbm.at[pl.ds(base + cb + HALF, HALF)])
                return 0
            lax.fori_loop(0, n_chunks, chunk_body, 0)

        run_lookup(h_hbm, etab, h_out)
        run_lookup(r_hbm, rtab, r_out)
        run_lookup(t_hbm, etab, t_out)

    return emb_kernel(h, r, t, entity_table, relation_table)
